# Initial kernel scaffold; baseline (speedup 1.0000x reference)
import jax
import jax.numpy as jnp
from jax.experimental import pallas as pl


def kernel(x, Adj, w1, w2, w3):
    raise NotImplementedError("write your pallas kernel here")


# speedup vs baseline: 1.0345x; 1.0345x over previous
import jax
import jax.numpy as jnp
from jax.experimental import pallas as pl
from jax.experimental.pallas import tpu as pltpu

_N = 10000
_D = 128
_BR = 400


def _pass1_kernel(adj_ref, x_ref, w1_ref, out_ref):
    h = jnp.dot(adj_ref[...], x_ref[...], preferred_element_type=jnp.float32)
    h = jnp.dot(h, w1_ref[...], preferred_element_type=jnp.float32)
    out_ref[...] = jnp.maximum(h, 0.0)


def _pass2_kernel(adj_ref, h_ref, w2_ref, w3_ref, out_ref):
    h = jnp.dot(adj_ref[...], h_ref[...], preferred_element_type=jnp.float32)
    h = jnp.maximum(jnp.dot(h, w2_ref[...], preferred_element_type=jnp.float32), 0.0)
    out_ref[...] = jnp.dot(h, w3_ref[...], preferred_element_type=jnp.float32)


def kernel(x, Adj, w1, w2, w3):
    grid = (_N // _BR,)
    params = pltpu.CompilerParams(
        dimension_semantics=(pltpu.GridDimensionSemantics.ARBITRARY,),
    )
    adj_spec = pl.BlockSpec((_BR, _N), lambda i: (i, 0))
    feat_spec = pl.BlockSpec((_N, _D), lambda i: (0, 0))
    w_spec = pl.BlockSpec((_D, _D), lambda i: (0, 0))
    out_spec = pl.BlockSpec((_BR, _D), lambda i: (i, 0))
    h1 = pl.pallas_call(
        _pass1_kernel,
        grid=grid,
        in_specs=[adj_spec, feat_spec, w_spec],
        out_specs=out_spec,
        out_shape=jax.ShapeDtypeStruct((_N, _D), jnp.float32),
        compiler_params=params,
    )(Adj, x, w1)
    out = pl.pallas_call(
        _pass2_kernel,
        grid=grid,
        in_specs=[adj_spec, feat_spec, w_spec, w_spec],
        out_specs=out_spec,
        out_shape=jax.ShapeDtypeStruct((_N, _D), jnp.float32),
        compiler_params=params,
    )(Adj, h1, w2, w3)
    return out
